# baseline (speedup 1.0000x reference)
import jax
import jax.numpy as jnp
from jax.experimental import pallas as pl
from jax.experimental.pallas import tpu as pltpu


_CHUNKS = 8


def _copy_body(u_ref, i_ref, out_u_ref, out_i_ref, sems):
    rows = u_ref.shape[0]
    step = rows // _CHUNKS
    copies = []
    for t, (src, dst) in enumerate(((u_ref, out_u_ref), (i_ref, out_i_ref))):
        for k in range(_CHUNKS):
            c = pltpu.make_async_copy(
                src.at[pl.ds(k * step, step), :],
                dst.at[pl.ds(k * step, step), :],
                sems.at[t * _CHUNKS + k],
            )
            c.start()
            copies.append(c)
    for c in copies:
        c.wait()


def kernel(user_history, user_table, item_table):
    del user_history
    n_rows, dim = user_table.shape
    wide_rows = 512
    wide_cols = (n_rows * dim) // wide_rows
    u = user_table.reshape(wide_rows, wide_cols)
    i = item_table.reshape(wide_rows, wide_cols)
    out_shapes = (
        jax.ShapeDtypeStruct((wide_rows, wide_cols), user_table.dtype),
        jax.ShapeDtypeStruct((wide_rows, wide_cols), item_table.dtype),
    )
    user_emb, item_emb = pl.pallas_call(
        _copy_body,
        out_shape=out_shapes,
        in_specs=[
            pl.BlockSpec(memory_space=pl.ANY),
            pl.BlockSpec(memory_space=pl.ANY),
        ],
        out_specs=(
            pl.BlockSpec(memory_space=pl.ANY),
            pl.BlockSpec(memory_space=pl.ANY),
        ),
        scratch_shapes=[pltpu.SemaphoreType.DMA((2 * _CHUNKS,))],
    )(u, i)
    return (user_emb.reshape(n_rows, dim), item_emb.reshape(n_rows, dim))
